# baseline (speedup 1.0000x reference)
import functools
import jax
import jax.numpy as jnp
from jax import lax
from jax.experimental import pallas as pl
from jax.experimental.pallas import tpu as pltpu

_C = 768
_CH = 384
_KO = 128
_HW = 4096
_GAP_THR = 1e-8


def _mean_tree(x):
    v = (x[:, 0:1024] + x[:, 1024:2048]) + (x[:, 2048:3072] + x[:, 3072:4096])
    s = [v[:, 128 * j:128 * j + 128] for j in range(8)]
    t = ((s[0] + s[4]) + (s[1] + s[5])) + ((s[2] + s[6]) + (s[3] + s[7]))
    return jnp.sum(t, axis=1) * (1.0 / 4096.0)


def _mean_guard_body(x1_ref, x2_ref, pidx_ref, gmin_ref):
    v = jnp.concatenate([_mean_tree(x1_ref[0]), _mean_tree(x2_ref[0])])
    vj = v[:, None]
    vc = v[None, :]
    ij = lax.broadcasted_iota(jnp.int32, (_C, _C), 0)
    ic = lax.broadcasted_iota(jnp.int32, (_C, _C), 1)
    beats = (vj > vc) | ((vj == vc) & (ij < ic))
    rank = jnp.sum(beats.astype(jnp.int32), axis=0)
    pr = lax.broadcasted_iota(jnp.int32, (_C, _C), 0)
    onehot = (rank[None, :] == pr).astype(jnp.float32)
    cf = ic.astype(jnp.float32)
    pidx_ref[0, 0, :] = jnp.sum(onehot * cf, axis=1).astype(jnp.int32)
    sortedv = jnp.sum(onehot * v[None, :], axis=1)
    gmin = jnp.min(sortedv[:-1] - sortedv[1:])
    gmin_ref[0, 0, :] = jnp.full((128,), gmin, jnp.float32)


def _route_body(pidx_ref, w_ref, x1_ref, x2_ref, out_ref, m_ref):
    j = pl.program_id(1)

    @pl.when(j == 0)
    def _build_m():
        pidx = pidx_ref[0, 0, :]
        ic = lax.broadcasted_iota(jnp.int32, (_CH, _C), 1)
        top = (pidx[:_CH, None] == ic).astype(jnp.float32)
        sel = (pidx[_CH:, None] == ic).astype(jnp.float32)
        m_ref[:_CH, :] = top
        m_ref[_CH:, :] = jnp.dot(w_ref[...], sel,
                                 preferred_element_type=jnp.float32)

    m = m_ref[...].astype(jnp.bfloat16)
    out_ref[0, :, :] = (
        jnp.dot(m[:, :_CH], x1_ref[0].astype(jnp.bfloat16),
                preferred_element_type=jnp.float32)
        + jnp.dot(m[:, _CH:], x2_ref[0].astype(jnp.bfloat16),
                  preferred_element_type=jnp.float32)
    )


def kernel(x1, x2, W):
    b = x1.shape[0]
    x1f = x1.reshape(b, _CH, _HW)
    x2f = x2.reshape(b, _CH, _HW)

    pidx_fast, gmin = pl.pallas_call(
        _mean_guard_body,
        grid=(b,),
        in_specs=[
            pl.BlockSpec((1, _CH, _HW), lambda i: (i, 0, 0)),
            pl.BlockSpec((1, _CH, _HW), lambda i: (i, 0, 0)),
        ],
        out_specs=[
            pl.BlockSpec((1, 1, _C), lambda i: (i, 0, 0)),
            pl.BlockSpec((1, 1, 128), lambda i: (i, 0, 0)),
        ],
        out_shape=[
            jax.ShapeDtypeStruct((b, 1, _C), jnp.int32),
            jax.ShapeDtypeStruct((b, 1, 128), jnp.float32),
        ],
        compiler_params=pltpu.CompilerParams(
            dimension_semantics=("parallel",)),
    )(x1f, x2f)
    safe = jnp.all(gmin[:, 0, 0] > _GAP_THR)

    def _fast(ops):
        return ops[0]

    def _slow(ops):
        _, xx1, xx2 = ops
        x = jnp.concatenate([xx1, xx2], axis=1)
        pooled = jnp.mean(x, axis=(2, 3))
        _, pidx = lax.top_k(pooled, _C)
        xs = jnp.take_along_axis(x, pidx[:, :, None, None], axis=1)
        pidx_b, xs_b = lax.optimization_barrier((pidx, xs))
        marker = (xs_b[:, :, 0, 0] * 0.0).astype(jnp.int32)
        return (pidx_b + marker)[:, None, :]

    pidx = lax.cond(safe, _fast, _slow, (pidx_fast, x1, x2))

    hwblk = 4096
    out = pl.pallas_call(
        _route_body,
        grid=(b, _HW // hwblk),
        in_specs=[
            pl.BlockSpec((1, 1, _C), lambda i, j: (i, 0, 0)),
            pl.BlockSpec((_KO, _CH), lambda i, j: (0, 0)),
            pl.BlockSpec((1, _CH, hwblk), lambda i, j: (i, 0, j)),
            pl.BlockSpec((1, _CH, hwblk), lambda i, j: (i, 0, j)),
        ],
        out_specs=pl.BlockSpec((1, _CH + _KO, hwblk), lambda i, j: (i, 0, j)),
        out_shape=jax.ShapeDtypeStruct((b, _CH + _KO, _HW), jnp.float32),
        scratch_shapes=[pltpu.VMEM((_CH + _KO, _C), jnp.float32)],
        compiler_params=pltpu.CompilerParams(
            dimension_semantics=("parallel", "arbitrary")),
    )(pidx, W, x1f, x2f)

    return out.reshape(b, _CH + _KO, 64, 64)


# speedup vs baseline: 1.0018x; 1.0018x over previous
import functools
import jax
import jax.numpy as jnp
from jax import lax
from jax.experimental import pallas as pl
from jax.experimental.pallas import tpu as pltpu

_C = 768
_CH = 384
_KO = 128
_HW = 4096
_GAP_THR = 1e-8


def _mean_tree(x):
    v = (x[:, 0:1024] + x[:, 1024:2048]) + (x[:, 2048:3072] + x[:, 3072:4096])
    s = [v[:, 128 * j:128 * j + 128] for j in range(8)]
    t = ((s[0] + s[4]) + (s[1] + s[5])) + ((s[2] + s[6]) + (s[3] + s[7]))
    return jnp.sum(t, axis=1) * (1.0 / 4096.0)


def _mean_guard_body(x1_ref, x2_ref, pidx_ref, gmin_ref):
    v = jnp.concatenate([_mean_tree(x1_ref[0]), _mean_tree(x2_ref[0])])
    vj = v[:, None]
    vc = v[None, :]
    ij = lax.broadcasted_iota(jnp.int32, (_C, _C), 0)
    ic = lax.broadcasted_iota(jnp.int32, (_C, _C), 1)
    beats = (vj > vc) | ((vj == vc) & (ij < ic))
    rank = jnp.sum(beats.astype(jnp.int32), axis=0)
    pr = lax.broadcasted_iota(jnp.int32, (_C, _C), 0)
    onehot = (rank[None, :] == pr).astype(jnp.float32)
    cf = ic.astype(jnp.float32)
    pidx_ref[0, 0, :] = jnp.sum(onehot * cf, axis=1).astype(jnp.int32)
    sortedv = jnp.sum(onehot * v[None, :], axis=1)
    gmin = jnp.min(sortedv[:-1] - sortedv[1:])
    gmin_ref[0, 0, :] = jnp.full((128,), gmin, jnp.float32)


def _route_body(pidx_ref, w_ref, x1_ref, x2_ref, out_ref, m_ref):
    j = pl.program_id(1)

    @pl.when(j == 0)
    def _build_m():
        pidx = pidx_ref[0, 0, :]
        ic = lax.broadcasted_iota(jnp.int32, (_CH, _C), 1)
        top = (pidx[:_CH, None] == ic).astype(jnp.float32)
        sel = (pidx[_CH:, None] == ic).astype(jnp.float32)
        m_ref[:_CH, :] = top
        m_ref[_CH:, :] = jnp.dot(w_ref[...], sel,
                                 preferred_element_type=jnp.float32)

    m = m_ref[...]
    out_ref[0, :, :] = (
        jnp.dot(m[:, :_CH], x1_ref[0], preferred_element_type=jnp.float32)
        + jnp.dot(m[:, _CH:], x2_ref[0], preferred_element_type=jnp.float32)
    )


def kernel(x1, x2, W):
    b = x1.shape[0]
    x1f = x1.reshape(b, _CH, _HW)
    x2f = x2.reshape(b, _CH, _HW)

    pidx_fast, gmin = pl.pallas_call(
        _mean_guard_body,
        grid=(b,),
        in_specs=[
            pl.BlockSpec((1, _CH, _HW), lambda i: (i, 0, 0)),
            pl.BlockSpec((1, _CH, _HW), lambda i: (i, 0, 0)),
        ],
        out_specs=[
            pl.BlockSpec((1, 1, _C), lambda i: (i, 0, 0)),
            pl.BlockSpec((1, 1, 128), lambda i: (i, 0, 0)),
        ],
        out_shape=[
            jax.ShapeDtypeStruct((b, 1, _C), jnp.int32),
            jax.ShapeDtypeStruct((b, 1, 128), jnp.float32),
        ],
        compiler_params=pltpu.CompilerParams(
            dimension_semantics=("parallel",)),
    )(x1f, x2f)
    safe = jnp.all(gmin[:, 0, 0] > _GAP_THR)

    def _fast(ops):
        return ops[0]

    def _slow(ops):
        _, xx1, xx2 = ops
        x = jnp.concatenate([xx1, xx2], axis=1)
        pooled = jnp.mean(x, axis=(2, 3))
        _, pidx = lax.top_k(pooled, _C)
        xs = jnp.take_along_axis(x, pidx[:, :, None, None], axis=1)
        pidx_b, xs_b = lax.optimization_barrier((pidx, xs))
        marker = (xs_b[:, :, 0, 0] * 0.0).astype(jnp.int32)
        return (pidx_b + marker)[:, None, :]

    pidx = lax.cond(safe, _fast, _slow, (pidx_fast, x1, x2))

    hwblk = 4096
    out = pl.pallas_call(
        _route_body,
        grid=(b, _HW // hwblk),
        in_specs=[
            pl.BlockSpec((1, 1, _C), lambda i, j: (i, 0, 0)),
            pl.BlockSpec((_KO, _CH), lambda i, j: (0, 0)),
            pl.BlockSpec((1, _CH, hwblk), lambda i, j: (i, 0, j)),
            pl.BlockSpec((1, _CH, hwblk), lambda i, j: (i, 0, j)),
        ],
        out_specs=pl.BlockSpec((1, _CH + _KO, hwblk), lambda i, j: (i, 0, j)),
        out_shape=jax.ShapeDtypeStruct((b, _CH + _KO, _HW), jnp.float32),
        scratch_shapes=[pltpu.VMEM((_CH + _KO, _C), jnp.float32)],
        compiler_params=pltpu.CompilerParams(
            dimension_semantics=("parallel", "arbitrary")),
    )(pidx, W, x1f, x2f)

    return out.reshape(b, _CH + _KO, 64, 64)
